# baseline (speedup 1.0000x reference)
import functools

import jax
import jax.numpy as jnp
from jax import lax
from jax.experimental import pallas as pl
from jax.experimental.pallas import tpu as pltpu
from jax.experimental.pallas import tpu_sc as plsc

B, S, D = 4, 4096, 1024
ROWS = B * S
NW = 32
ROWS_PER_W = ROWS // NW
CH = 32
NCH = ROWS_PER_W // CH
NL = 16


def _body(x_hbm, pe_hbm, pos_hbm, out_hbm, xbuf, pebuf, idx_v, sem):
    c = lax.axis_index("c")
    s = lax.axis_index("s")
    wid = s * 2 + c

    def chunk(i, carry):
        base = wid * ROWS_PER_W + i * CH
        pe_base = lax.rem(base, S)
        pltpu.sync_copy(x_hbm.at[pl.ds(base, CH)], xbuf)
        pltpu.sync_copy(pos_hbm.at[pl.ds(pe_base, CH)], idx_v)
        pltpu.async_copy(pe_hbm.at[idx_v], pebuf, sem).wait()

        def add_row(r, carry2):
            for j in range(D // NL):
                sl = pl.ds(j * NL, NL)
                xbuf[r, sl] = xbuf[r, sl] + pebuf[r, sl]
            return carry2

        lax.fori_loop(0, CH, add_row, 0)
        pltpu.sync_copy(xbuf, out_hbm.at[pl.ds(base, CH)])
        return carry

    lax.fori_loop(0, NCH, chunk, 0)


@jax.jit
def kernel(x, pos_embed):
    x2d = x.reshape(ROWS, D)
    positions = jnp.arange(S, dtype=jnp.int32)
    mesh = plsc.VectorSubcoreMesh(core_axis_name="c", subcore_axis_name="s")
    run = functools.partial(
        pl.kernel,
        mesh=mesh,
        out_type=jax.ShapeDtypeStruct((ROWS, D), jnp.float32),
        scratch_types=[
            pltpu.VMEM((CH, D), jnp.float32),
            pltpu.VMEM((CH, D), jnp.float32),
            pltpu.VMEM((CH,), jnp.int32),
            pltpu.SemaphoreType.DMA,
        ],
    )(_body)
    out2d = run(x2d, pos_embed, positions)
    return out2d.reshape(B, S, D)


# speedup vs baseline: 3.6548x; 3.6548x over previous
import functools

import jax
import jax.numpy as jnp
from jax import lax
from jax.experimental import pallas as pl
from jax.experimental.pallas import tpu as pltpu

B, S, D = 4, 4096, 1024
ROWS = B * S
CHR = 512
NC = ROWS // CHR
NBUF = 4
PECH = S // CHR
RSUB = 64


def _body(x_hbm, pe_hbm, o_hbm, xbuf, obuf, pebuf, in_sems, out_sems, pe_sems):
    def in_copy(i, slot):
        return pltpu.make_async_copy(
            x_hbm.at[pl.ds(i * CHR, CHR)], xbuf.at[slot], in_sems.at[slot])

    def out_copy(i, slot):
        return pltpu.make_async_copy(
            obuf.at[slot], o_hbm.at[pl.ds(i * CHR, CHR)], out_sems.at[slot])

    def pe_copy(j):
        return pltpu.make_async_copy(
            pe_hbm.at[pl.ds(j * CHR, CHR)], pebuf.at[j], pe_sems.at[j])

    for k in range(NBUF):
        in_copy(k, k).start()
        pe_copy(k).start()
    for j in range(NBUF, PECH):
        pe_copy(j).start()

    def step(i, carry):
        slot = lax.rem(i, NBUF)
        pe_j = lax.rem(i, PECH)
        in_copy(i, slot).wait()

        @pl.when(i < PECH)
        def _():
            pe_copy(pe_j).wait()

        @pl.when(i >= NBUF)
        def _():
            out_copy(i - NBUF, slot).wait()

        def add_sub(k, c2):
            rs = pl.ds(k * RSUB, RSUB)
            obuf[slot, rs] = xbuf[slot, rs] + pebuf[pe_j, rs]
            return c2

        lax.fori_loop(0, CHR // RSUB, add_sub, 0)
        out_copy(i, slot).start()

        @pl.when(i + NBUF < NC)
        def _():
            in_copy(i + NBUF, slot).start()

        return carry

    lax.fori_loop(0, NC, step, 0)

    for i in range(NC - NBUF, NC):
        out_copy(i, i % NBUF).wait()


@jax.jit
def kernel(x, pos_embed):
    x2d = x.reshape(ROWS, D)
    out2d = pl.pallas_call(
        _body,
        in_specs=[
            pl.BlockSpec(memory_space=pl.ANY),
            pl.BlockSpec(memory_space=pl.ANY),
        ],
        out_specs=pl.BlockSpec(memory_space=pl.ANY),
        out_shape=jax.ShapeDtypeStruct((ROWS, D), jnp.float32),
        scratch_shapes=[
            pltpu.VMEM((NBUF, CHR, D), jnp.float32),
            pltpu.VMEM((NBUF, CHR, D), jnp.float32),
            pltpu.VMEM((PECH, CHR, D), jnp.float32),
            pltpu.SemaphoreType.DMA((NBUF,)),
            pltpu.SemaphoreType.DMA((NBUF,)),
            pltpu.SemaphoreType.DMA((PECH,)),
        ],
    )(x2d, pos_embed)
    return out2d.reshape(B, S, D)
